# Initial kernel scaffold; baseline (speedup 1.0000x reference)
import jax
import jax.numpy as jnp
from jax.experimental import pallas as pl


def kernel(x, edge_index, edge_weight, batch, W1, b1, W2, b2, W3, b3, gamma, beta, Wl, bl):
    raise NotImplementedError("write your pallas kernel here")


# speedup vs baseline: 2.6478x; 2.6478x over previous
import functools
import jax
import jax.numpy as jnp
from jax import lax
from jax.experimental import pallas as pl
from jax.experimental.pallas import tpu as pltpu
from jax.experimental.pallas import tpu_sc as plsc

N = 10000
E = 320000
H = 128
G = 64
W2P = 256
W3P = 384
NP = 10240
NPT = NP // 16
BLK = 128
NW = 32
NBLK = (E + NW * BLK - 1) // (NW * BLK)
EPT = NBLK * BLK
EP = NW * EPT
RB = 2560

def _mesh():
    return plsc.VectorSubcoreMesh(core_axis_name="c", subcore_axis_name="s")


def _seg_body(table, src3, dst3, ew3, zrow, out, src_v, dst_v, ew_v, rows,
              acc_sh, sem):
    cid = lax.axis_index("c")
    sid = lax.axis_index("s")
    wid = sid * 2 + cid
    pltpu.sync_copy(zrow, acc_sh.at[pl.ds(sid * NPT, NPT)])
    plsc.subcore_barrier()
    pltpu.sync_copy(src3.at[wid], src_v)
    pltpu.sync_copy(dst3.at[wid], dst_v)
    pltpu.sync_copy(ew3.at[wid], ew_v)

    def block(b, _):
        pltpu.async_copy(table.at[src_v.at[b]], rows, sem).wait()

        def group(g, carry):
            ewv = ew_v[b, pl.ds(g * 16, 16)]
            for l in range(16):
                wgt = ewv[l]
                for j in range(H // 16):
                    sl = pl.ds(j * 16, 16)
                    rows[g * 16 + l, sl] = rows[g * 16 + l, sl] * wgt
            return carry

        lax.fori_loop(0, BLK // 16, group, 0)
        pltpu.sync_copy(rows, acc_sh.at[dst_v.at[b]], add=True)
        return _

    lax.fori_loop(0, NBLK, block, 0)
    plsc.subcore_barrier()
    pltpu.sync_copy(acc_sh.at[pl.ds(sid * NPT, NPT)],
                    out.at[cid].at[pl.ds(sid * NPT, NPT)])


def _sc_segsum(table, src3, dst3, ew3):
    zrow = jnp.zeros((NPT, H), jnp.float32)
    f = pl.kernel(
        _seg_body,
        out_type=jax.ShapeDtypeStruct((2, NP, H), jnp.float32),
        mesh=_mesh(),
        scratch_types=[
            pltpu.VMEM((NBLK, BLK), jnp.int32),
            pltpu.VMEM((NBLK, BLK), jnp.int32),
            pltpu.VMEM((NBLK, BLK), jnp.float32),
            pltpu.VMEM((BLK, H), jnp.float32),
            pltpu.VMEM_SHARED((NP, H), jnp.float32),
            pltpu.SemaphoreType.DMA,
        ],
    )
    return f(table, src3, dst3, ew3, zrow)


def _pool_body(pc0, pc1, pc2, ones_t, batch3, zgg, pout,
               batch2_v, bidx_v, rows, pool_sh, sem):
    cid = lax.axis_index("c")
    sid = lax.axis_index("s")
    my0 = sid * 4 * G
    pltpu.sync_copy(zgg, pool_sh.at[pl.ds(my0, 4 * G)])
    pltpu.sync_copy(batch3.at[sid], batch2_v)

    def scat(k, c, rows):
        base_row = my0 + c * G
        for gch in range(BLK // 16):
            sl = pl.ds(gch * 16, 16)
            bidx_v[sl] = batch2_v[k, sl] + base_row
        pltpu.sync_copy(rows, pool_sh.at[bidx_v], add=True)

    for c, pc in enumerate((pc0, pc1, pc2)):
        def pblock(k, _, pc=pc, c=c):
            pltpu.sync_copy(pc.at[cid].at[pl.ds(sid * NPT + k * BLK, BLK)],
                            rows)
            scat(k, c, rows)
            return _

        lax.fori_loop(0, NPT // BLK, pblock, 0)

    @pl.when(cid == 0)
    def _():
        def qblock(k, _):
            pltpu.sync_copy(ones_t.at[pl.ds(sid * NPT + k * BLK, BLK)], rows)
            scat(k, 3, rows)
            return _

        lax.fori_loop(0, NPT // BLK, qblock, 0)

    pltpu.sync_copy(pool_sh.at[pl.ds(my0, 4 * G)],
                    out_slice(pout, cid, my0))


def out_slice(pout, cid, my0):
    return pout.at[cid].at[pl.ds(my0, 4 * G)]


def _sc_pool(pc0, pc1, pc2, ones_t, batch_pad):
    zgg = jnp.zeros((4 * G, H), jnp.float32)
    batch3 = batch_pad.reshape(16, NPT // BLK, BLK)
    f = pl.kernel(
        _pool_body,
        out_type=jax.ShapeDtypeStruct((2, 16 * 4 * G, H), jnp.float32),
        mesh=_mesh(),
        scratch_types=[
            pltpu.VMEM((NPT // BLK, BLK), jnp.int32),
            pltpu.VMEM((BLK,), jnp.int32),
            pltpu.VMEM((BLK, H), jnp.float32),
            pltpu.VMEM_SHARED((16 * 4 * G, H), jnp.float32),
            pltpu.SemaphoreType.DMA,
        ],
    )
    return f(pc0, pc1, pc2, ones_t, batch3, zgg)


def _denseA_body(pa_ref, x_ref, w1_ref, b1_ref, w2_ref, h1_ref, hw2a_ref,
                 hw2b_ref):
    s0 = (pa_ref[0] + pa_ref[1])[:, :1]
    h1 = jax.nn.relu(s0 * w1_ref[...] + b1_ref[...][None, :])
    h1_ref[...] = h1
    hcat = jnp.concatenate(
        [h1, x_ref[...], jnp.zeros((RB, W2P - H - 1), jnp.float32)], axis=1)
    hw2 = jnp.dot(hcat.astype(jnp.bfloat16), w2_ref[...],
                  preferred_element_type=jnp.float32)
    hw2a_ref[...] = hw2[:, :H]
    hw2b_ref[...] = hw2[:, H:]


def _denseA(pa, xp, W1, b1, W2p):
    grid = NP // RB
    return pl.pallas_call(
        _denseA_body,
        grid=(grid,),
        in_specs=[
            pl.BlockSpec((2, RB, H), lambda i: (0, i, 0)),
            pl.BlockSpec((RB, 1), lambda i: (i, 0)),
            pl.BlockSpec((1, H), lambda i: (0, 0)),
            pl.BlockSpec((H,), lambda i: (0,)),
            pl.BlockSpec((W2P, W2P), lambda i: (0, 0)),
        ],
        out_specs=[
            pl.BlockSpec((RB, H), lambda i: (i, 0)),
            pl.BlockSpec((RB, H), lambda i: (i, 0)),
            pl.BlockSpec((RB, W2P - H), lambda i: (i, 0)),
        ],
        out_shape=[
            jax.ShapeDtypeStruct((NP, H), jnp.float32),
            jax.ShapeDtypeStruct((NP, H), jnp.float32),
            jax.ShapeDtypeStruct((NP, W2P - H), jnp.float32),
        ],
    )(pa, xp, W1, b1, W2p)


def _denseB_body(pba_ref, pbb_ref, h1_ref, b2_ref, w3_ref, c0_ref, c1_ref,
                 c2_ref):
    agg_a = pba_ref[0] + pba_ref[1]
    agg_b = pbb_ref[0] + pbb_ref[1]
    agg = jnp.concatenate([agg_a, agg_b], axis=1)
    h2 = jax.nn.relu(agg + b2_ref[...][None, :])
    hcat = jnp.concatenate(
        [h2[:, : H + 1], h1_ref[...],
         jnp.zeros((RB, W3P - 2 * H - 1), jnp.float32)], axis=1)
    hw3 = jnp.dot(hcat.astype(jnp.bfloat16), w3_ref[...],
                  preferred_element_type=jnp.float32)
    c0_ref[...] = hw3[:, :H]
    c1_ref[...] = hw3[:, H:2 * H]
    c2_ref[...] = hw3[:, 2 * H:]


def _denseB(pba, pbb, h1, b2p, W3p):
    grid = NP // RB
    return pl.pallas_call(
        _denseB_body,
        grid=(grid,),
        in_specs=[
            pl.BlockSpec((2, RB, H), lambda i: (0, i, 0)),
            pl.BlockSpec((2, RB, H), lambda i: (0, i, 0)),
            pl.BlockSpec((RB, H), lambda i: (i, 0)),
            pl.BlockSpec((W2P,), lambda i: (0,)),
            pl.BlockSpec((W3P, W3P), lambda i: (0, 0)),
        ],
        out_specs=[
            pl.BlockSpec((RB, H), lambda i: (i, 0)),
            pl.BlockSpec((RB, H), lambda i: (i, 0)),
            pl.BlockSpec((RB, W3P - 2 * H), lambda i: (i, 0)),
        ],
        out_shape=[
            jax.ShapeDtypeStruct((NP, H), jnp.float32),
            jax.ShapeDtypeStruct((NP, H), jnp.float32),
            jax.ShapeDtypeStruct((NP, W3P - 2 * H), jnp.float32),
        ],
    )(pba, pbb, h1, b2p, W3p)


def _final_body(pout_ref, b3_ref, gamma_ref, beta_ref,
                wl_ref, bl_ref, out_ref):
    full = pout_ref[...].reshape(2, 16, 4, G, H)
    p = jnp.sum(jnp.sum(full, axis=0), axis=0)
    feats = jnp.concatenate([p[0], p[1], p[2]], axis=1)
    sums = feats[:, : 2 * H + 1]
    cnts = p[3][:, :1]
    sums = sums + cnts * b3_ref[...][None, :]
    g = sums / jnp.maximum(cnts, 1.0)
    mean = jnp.mean(g, axis=0)
    var = jnp.mean((g - mean) ** 2, axis=0)
    g = (g - mean) * jax.lax.rsqrt(var + 1e-5) * gamma_ref[...][None, :] \
        + beta_ref[...][None, :]
    out_ref[...] = jnp.dot(g, wl_ref[...], preferred_element_type=jnp.float32) \
        + bl_ref[...][None, :]


def _final(pout, b3, gamma, beta, Wl, bl):
    return pl.pallas_call(
        _final_body,
        out_shape=jax.ShapeDtypeStruct((G, 1), jnp.float32),
    )(pout, b3, gamma, beta, Wl, bl)


def kernel(x, edge_index, edge_weight, batch, W1, b1, W2, b2, W3, b3, gamma, beta, Wl, bl):
    src = edge_index[0].astype(jnp.int32)
    dst = edge_index[1].astype(jnp.int32)
    src3 = jnp.pad(src, (0, EP - E)).reshape(NW, NBLK, BLK)
    dst3 = jnp.pad(dst, (0, EP - E)).reshape(NW, NBLK, BLK)
    ew3 = jnp.pad(edge_weight, (0, EP - E)).reshape(NW, NBLK, BLK)
    batch_pad = jnp.pad(batch.astype(jnp.int32), (0, NP - N))
    xp = jnp.pad(x, ((0, NP - N), (0, 0)))
    x128 = jnp.pad(x, ((0, NP - N), (0, H - 1)))
    row_lt_n = (jnp.arange(NP) < N).astype(jnp.float32)[:, None]
    ones_t = jnp.pad(row_lt_n, ((0, 0), (0, H - 1)))

    W2p = jnp.pad(W2, ((0, W2P - (H + 1)), (0, W2P - (H + 1)))) \
        .astype(jnp.bfloat16)
    b2p = jnp.pad(b2, (0, W2P - (H + 1)))
    W3p = jnp.pad(W3, ((0, W3P - (2 * H + 1)), (0, W3P - (2 * H + 1)))) \
        .astype(jnp.bfloat16)

    pa = _sc_segsum(x128, src3, dst3, ew3)
    h1, hw2a, hw2b = _denseA(pa, xp, W1, b1, W2p)
    pba = _sc_segsum(hw2a, src3, dst3, ew3)
    pbb = _sc_segsum(hw2b, src3, dst3, ew3)
    c0, c1, c2 = _denseB(pba, pbb, h1, b2p, W3p)
    pc0 = _sc_segsum(c0, src3, dst3, ew3)
    pc1 = _sc_segsum(c1, src3, dst3, ew3)
    pc2 = _sc_segsum(c2, src3, dst3, ew3)
    pout = _sc_pool(pc0, pc1, pc2, ones_t, batch_pad)
    return _final(pout, b3, gamma, beta, Wl, bl)
